# Initial kernel scaffold; baseline (speedup 1.0000x reference)
import math

import jax
import jax.numpy as jnp
from jax import lax
from jax.experimental import pallas as pl
from jax.experimental.pallas import tpu as pltpu
from jax.experimental.pallas import tpu_sc as plsc

_N_ATOMS = 100000
_N_EDGES = 6400000
_FEATS = 8
_NC = 2
_NS = 16
_NW = _NC * _NS
_PER_W = _N_EDGES // _NW
_E = 800
_CHUNKS = _PER_W // _E
_VPC = _E // 16
_ROWS_PER_TILE = _N_ATOMS // _NS

_BETA = 4.0 ** (1.0 / 7.0)
_CK = [2.0 * _BETA**k * math.log(2.0) for k in range(_FEATS)]
_LOG2C = 2.0 / math.log(2.0)


def _sc_body(r_hbm, nl_hbm, z_hbm, out_hbm, z_v, i_v, j_v, r_v, fe_v, acc_sh):
    cid = lax.axis_index("c")
    sid = lax.axis_index("s")
    wid = cid * _NS + sid
    lanes = lax.iota(jnp.int32, 16)
    zero16 = jnp.zeros((16,), jnp.float32)

    pltpu.sync_copy(z_hbm, z_v)

    def _zero(t, carry):
        n = t * 16 + lanes
        plsc.store_scatter(fe_v, [n >> 3, n & 7], zero16)
        return carry

    lax.fori_loop(0, (_E * _FEATS) // 16, _zero, 0)
    row0 = sid * _ROWS_PER_TILE
    for t in range(_ROWS_PER_TILE // _E):
        pltpu.sync_copy(fe_v, acc_sh.at[pl.ds(row0 + t * _E, _E), :])
    _TAIL = _ROWS_PER_TILE % _E
    if _TAIL:
        pltpu.sync_copy(
            fe_v.at[pl.ds(0, _TAIL), :],
            acc_sh.at[pl.ds(row0 + _ROWS_PER_TILE - _TAIL, _TAIL), :],
        )
    plsc.subcore_barrier()

    def _chunk(c, carry):
        base = wid * _PER_W + c * _E
        pltpu.sync_copy(nl_hbm.at[0, pl.ds(base, _E)], i_v)
        pltpu.sync_copy(nl_hbm.at[1, pl.ds(base, _E)], j_v)
        pltpu.sync_copy(r_hbm.at[pl.ds(base, _E)], r_v)

        def _vec(o, c2):
            s = o * 16
            i16 = i_v[pl.ds(s, 16)]
            j16 = j_v[pl.ds(s, 16)]
            r16 = r_v[pl.ds(s, 16)]
            zi = plsc.load_gather(z_v, [i16])
            zj = plsc.load_gather(z_v, [j16])
            msk = jnp.where((zi == 1) & (zj == 2), 1.0, 0.0).astype(jnp.float32)
            f = jnp.maximum(2.0 * (1.0 - r16 * 0.2), 0.0)
            bits = plsc.bitcast(f, jnp.int32)
            e = ((bits >> 23) - 127).astype(jnp.float32)
            m = plsc.bitcast((bits & 0x7FFFFF) | 0x3F800000, jnp.float32)
            t = (m - 1.0) / (m + 1.0)
            t2 = t * t
            log2m = _LOG2C * (
                t * (1.0 + t2 * (1.0 / 3.0 + t2 * (0.2 + t2 * (1.0 / 7.0))))
            )
            u = e + log2m
            rowi = s + lanes
            for k in range(_FEATS):
                y = msk * jnp.exp(_CK[k] * u)
                plsc.store_scatter(fe_v, [rowi, jnp.full((16,), k, jnp.int32)], y)
            return c2

        lax.fori_loop(0, _VPC, _vec, 0)
        pltpu.sync_copy(fe_v, acc_sh.at[i_v], add=True)
        return carry

    lax.fori_loop(0, _CHUNKS, _chunk, 0)
    plsc.subcore_barrier()
    pltpu.sync_copy(
        acc_sh.at[pl.ds(row0, _ROWS_PER_TILE), :],
        out_hbm.at[cid, pl.ds(row0, _ROWS_PER_TILE), :],
    )


_sc_call = pl.kernel(
    _sc_body,
    out_type=jax.ShapeDtypeStruct((_NC, _N_ATOMS, _FEATS), jnp.float32),
    mesh=plsc.VectorSubcoreMesh(core_axis_name="c", subcore_axis_name="s"),
    scratch_types=[
        pltpu.VMEM((_N_ATOMS,), jnp.int32),
        pltpu.VMEM((_E,), jnp.int32),
        pltpu.VMEM((_E,), jnp.int32),
        pltpu.VMEM((_E,), jnp.float32),
        pltpu.VMEM((_E, _FEATS), jnp.float32),
        pltpu.VMEM_SHARED((_N_ATOMS, _FEATS), jnp.float32),
    ],
)


def _combine_body(p_ref, o_ref):
    o_ref[...] = p_ref[0] + p_ref[1]


def _combine(partial):
    p3 = partial.reshape(_NC, 6250, 128)
    out = pl.pallas_call(
        _combine_body,
        grid=(5,),
        in_specs=[pl.BlockSpec((_NC, 1250, 128), lambda i: (0, i, 0))],
        out_specs=pl.BlockSpec((1250, 128), lambda i: (i, 0)),
        out_shape=jax.ShapeDtypeStruct((6250, 128), jnp.float32),
    )(p3)
    return out.reshape(_N_ATOMS, _FEATS)


def kernel(r, neighbour_list, Z):
    partial = _sc_call(r, neighbour_list, Z)
    return _combine(partial)


# speedup vs baseline: 124.0348x; 124.0348x over previous
import math

import jax
import jax.numpy as jnp
from jax import lax
from jax.experimental import pallas as pl
from jax.experimental.pallas import tpu as pltpu
from jax.experimental.pallas import tpu_sc as plsc

_N_ATOMS = 100000
_N_EDGES = 6400000
_FEATS = 8
_NC = 2
_NS = 16
_NW = _NC * _NS
_PER_W = _N_EDGES // _NW
_E = 800
_CHUNKS = _PER_W // _E
_VPC = _E // 16
_ATOMS_PAD = 100352
_ROWS_PER_TILE = _ATOMS_PAD // _NS
_ZP_WORDS = _ATOMS_PAD // 16
_ZPW_PER_TILE = _ZP_WORDS // _NS

_BETA = 4.0 ** (1.0 / 7.0)
_CK = [2.0 * _BETA**k * math.log(2.0) for k in range(_FEATS)]
_LOG2C = 2.0 / math.log(2.0)


def _vgather(v, idx):
    return lax.gather(
        v,
        idx[:, None],
        lax.GatherDimensionNumbers(
            offset_dims=(), collapsed_slice_dims=(0,), start_index_map=(0,)
        ),
        (1,),
        mode=lax.GatherScatterMode.PROMISE_IN_BOUNDS,
    )


def _sc_body(r_hbm, nl_hbm, z_hbm, out_hbm, zp_v, i_v, j_v, r_v, fe_v, zp_sh, acc_sh):
    cid = lax.axis_index("c")
    sid = lax.axis_index("s")
    wid = cid * _NS + sid
    lanes = lax.iota(jnp.int32, 16)
    zero16 = jnp.zeros((16,), jnp.float32)

    a0 = sid * _ROWS_PER_TILE
    lane0 = lanes == 0
    zvec = lanes >> 4

    for c in range(8):
        cs = 800 if c < 7 else _ROWS_PER_TILE - 7 * 800
        pltpu.sync_copy(z_hbm.at[pl.ds(a0 + c * 800, cs)], i_v.at[pl.ds(0, cs)])

        def _pack(t, carry, _c=c, _cs=cs):
            z16 = i_v[pl.ds(t * 16, 16)]
            code = jnp.where(z16 == 1, 1, 0) | jnp.where(z16 == 2, 2, 0)
            s = code << (lanes * 2)
            s = s | _vgather(s, lanes ^ 1)
            s = s | _vgather(s, lanes ^ 2)
            s = s | _vgather(s, lanes ^ 4)
            s = s | _vgather(s, lanes ^ 8)
            w = (a0 + _c * 800 + t * 16) >> 4
            plsc.store_scatter(zp_v, [zvec + w], s, mask=lane0)
            return carry

        lax.fori_loop(0, cs // 16, _pack, 0)

    pltpu.sync_copy(
        zp_v.at[pl.ds(sid * _ZPW_PER_TILE, _ZPW_PER_TILE)],
        zp_sh.at[pl.ds(sid * _ZPW_PER_TILE, _ZPW_PER_TILE)],
    )

    def _zero(t, carry):
        n = t * 16 + lanes
        plsc.store_scatter(fe_v, [n >> 3, n & 7], zero16)
        return carry

    lax.fori_loop(0, (_E * _FEATS) // 16, _zero, 0)
    for t in range(_ROWS_PER_TILE // _E):
        pltpu.sync_copy(fe_v, acc_sh.at[pl.ds(a0 + t * _E, _E), :])
    _TAIL = _ROWS_PER_TILE % _E
    if _TAIL:
        pltpu.sync_copy(
            fe_v.at[pl.ds(0, _TAIL), :],
            acc_sh.at[pl.ds(a0 + _ROWS_PER_TILE - _TAIL, _TAIL), :],
        )

    plsc.subcore_barrier()
    pltpu.sync_copy(zp_sh, zp_v)

    def _chunk(c, carry):
        base = wid * _PER_W + c * _E
        pltpu.sync_copy(nl_hbm.at[pl.ds(base, _E)], i_v)
        pltpu.sync_copy(nl_hbm.at[pl.ds(_N_EDGES + base, _E)], j_v)
        pltpu.sync_copy(r_hbm.at[pl.ds(base, _E)], r_v)

        def _vec(o, c2):
            s = o * 16
            i16 = i_v[pl.ds(s, 16)]
            j16 = j_v[pl.ds(s, 16)]
            r16 = r_v[pl.ds(s, 16)]
            wi = plsc.load_gather(zp_v, [i16 >> 4])
            wj = plsc.load_gather(zp_v, [j16 >> 4])
            bi = (wi >> ((i16 & 15) * 2)) & 1
            bj = (wj >> ((j16 & 15) * 2 + 1)) & 1
            f = jnp.maximum(2.0 * (1.0 - r16 * 0.2), 0.0)
            bits = plsc.bitcast(f, jnp.int32)
            e = ((bits >> 23) - 127).astype(jnp.float32)
            m = plsc.bitcast((bits & 0x7FFFFF) | 0x3F800000, jnp.float32)
            t = (m - 1.0) / (m + 1.0)
            t2 = t * t
            log2m = _LOG2C * (
                t * (1.0 + t2 * (1.0 / 3.0 + t2 * (0.2 + t2 * (1.0 / 7.0))))
            )
            u = e + log2m
            u = jnp.where((bi & bj) == 1, u, -1000.0)
            rowi = s + lanes
            for k in range(_FEATS):
                y = jnp.exp(_CK[k] * u)
                plsc.store_scatter(fe_v, [rowi, jnp.full((16,), k, jnp.int32)], y)
            return c2

        lax.fori_loop(0, _VPC, _vec, 0)
        pltpu.sync_copy(fe_v, acc_sh.at[i_v], add=True)
        return carry

    lax.fori_loop(0, _CHUNKS, _chunk, 0)
    plsc.subcore_barrier()
    pltpu.sync_copy(
        acc_sh.at[pl.ds(a0, _ROWS_PER_TILE), :],
        out_hbm.at[cid, pl.ds(a0, _ROWS_PER_TILE), :],
    )


_sc_call = pl.kernel(
    _sc_body,
    out_type=jax.ShapeDtypeStruct((_NC, _ATOMS_PAD, _FEATS), jnp.float32),
    mesh=plsc.VectorSubcoreMesh(core_axis_name="c", subcore_axis_name="s"),
    compiler_params=pltpu.CompilerParams(
        needs_layout_passes=False, use_tc_tiling_on_sc=False
    ),
    scratch_types=[
        pltpu.VMEM((_ZP_WORDS,), jnp.int32),
        pltpu.VMEM((_E,), jnp.int32),
        pltpu.VMEM((_E,), jnp.int32),
        pltpu.VMEM((_E,), jnp.float32),
        pltpu.VMEM((_E, _FEATS), jnp.float32),
        pltpu.VMEM_SHARED((_ZP_WORDS,), jnp.int32),
        pltpu.VMEM_SHARED((_ATOMS_PAD, _FEATS), jnp.float32),
    ],
)


def _combine_body(p_ref, o_ref):
    o_ref[...] = p_ref[0] + p_ref[1]


def _combine(partial):
    p3 = partial.reshape(_NC, _ATOMS_PAD * _FEATS // 128, 128)
    out = pl.pallas_call(
        _combine_body,
        out_shape=jax.ShapeDtypeStruct((_ATOMS_PAD * _FEATS // 128, 128), jnp.float32),
    )(p3)
    return out.reshape(_ATOMS_PAD, _FEATS)[:_N_ATOMS]


def kernel(r, neighbour_list, Z):
    z_pad = jnp.concatenate([Z, jnp.zeros((_ATOMS_PAD - _N_ATOMS,), jnp.int32)])
    partial = _sc_call(r, neighbour_list.reshape(2 * _N_EDGES), z_pad)
    return _combine(partial)


# speedup vs baseline: 210.2318x; 1.6949x over previous
import math

import jax
import jax.numpy as jnp
from jax import lax
from jax.experimental import pallas as pl
from jax.experimental.pallas import tpu as pltpu
from jax.experimental.pallas import tpu_sc as plsc

_N_ATOMS = 100000
_N_EDGES = 6400000
_FEATS = 8
_NC = 2
_NS = 16
_NW = _NC * _NS
_PER_W = _N_EDGES // _NW
_E = 2000
_CHUNKS = _PER_W // _E
_VPC = _E // 16
_B = 128
_ATOMS_PAD = 100352
_ROWS_PER_TILE = _ATOMS_PAD // _NS
_ZP_WORDS = _ATOMS_PAD // 16
_ZPW_PER_TILE = _ZP_WORDS // _NS

_BETA = 4.0 ** (1.0 / 7.0)
_CK = [2.0 * _BETA**k * math.log(2.0) for k in range(_FEATS)]
_LOG2C = 2.0 / math.log(2.0)


def _vgather(v, idx):
    return lax.gather(
        v,
        idx[:, None],
        lax.GatherDimensionNumbers(
            offset_dims=(), collapsed_slice_dims=(0,), start_index_map=(0,)
        ),
        (1,),
        mode=lax.GatherScatterMode.PROMISE_IN_BOUNDS,
    )


def _log2(f):
    bits = plsc.bitcast(f, jnp.int32)
    e = ((bits >> 23) - 127).astype(jnp.float32)
    m = plsc.bitcast((bits & 0x7FFFFF) | 0x3F800000, jnp.float32)
    t = (m - 1.0) / (m + 1.0)
    t2 = t * t
    return e + _LOG2C * (
        t * (1.0 + t2 * (1.0 / 3.0 + t2 * (0.2 + t2 * (1.0 / 7.0))))
    )


def _sc_body(
    r_hbm, nl_hbm, z_hbm, out_hbm,
    zp_v, i_v, j_v, r_v, ik_v, rk_v, fe_v, zp_sh, acc_sh,
):
    cid = lax.axis_index("c")
    sid = lax.axis_index("s")
    wid = cid * _NS + sid
    lanes = lax.iota(jnp.int32, 16)
    zero16 = jnp.zeros((16,), jnp.float32)
    zeroi16 = lanes >> 4
    five16 = zero16 + 5.0
    lane0 = lanes == 0

    a0 = sid * _ROWS_PER_TILE

    for c in range(4):
        cs = _E if c < 3 else _ROWS_PER_TILE - 3 * _E
        pltpu.sync_copy(z_hbm.at[pl.ds(a0 + c * _E, cs)], i_v.at[pl.ds(0, cs)])

        def _pack(t, carry, _c=c):
            z16 = i_v[pl.ds(t * 16, 16)]
            code = jnp.where(z16 == 1, 1, 0) | jnp.where(z16 == 2, 2, 0)
            s = code << (lanes * 2)
            s = s | _vgather(s, lanes ^ 1)
            s = s | _vgather(s, lanes ^ 2)
            s = s | _vgather(s, lanes ^ 4)
            s = s | _vgather(s, lanes ^ 8)
            w = (a0 + _c * _E + t * 16) >> 4
            plsc.store_scatter(zp_v, [zeroi16 + w], s, mask=lane0)
            return carry

        lax.fori_loop(0, cs // 16, _pack, 0)

    pltpu.sync_copy(
        zp_v.at[pl.ds(sid * _ZPW_PER_TILE, _ZPW_PER_TILE)],
        zp_sh.at[pl.ds(sid * _ZPW_PER_TILE, _ZPW_PER_TILE)],
    )

    def _zero(t, carry):
        n = t * 16 + lanes
        plsc.store_scatter(fe_v, [n >> 3, n & 7], zero16)
        return carry

    lax.fori_loop(0, (_B * _FEATS) // 16, _zero, 0)
    for t in range(_ROWS_PER_TILE // _B):
        pltpu.sync_copy(fe_v, acc_sh.at[pl.ds(a0 + t * _B, _B), :])

    plsc.subcore_barrier()
    pltpu.sync_copy(zp_sh, zp_v)

    def _chunk(c, carry):
        base = wid * _PER_W + c * _E
        pltpu.sync_copy(nl_hbm.at[pl.ds(base, _E)], i_v)
        pltpu.sync_copy(nl_hbm.at[pl.ds(_N_EDGES + base, _E)], j_v)
        pltpu.sync_copy(r_hbm.at[pl.ds(base, _E)], r_v)

        def _vec(o, off):
            s = o * 16
            i16 = i_v[pl.ds(s, 16)]
            j16 = j_v[pl.ds(s, 16)]
            r16 = r_v[pl.ds(s, 16)]
            wi = plsc.load_gather(zp_v, [i16 >> 4])
            wj = plsc.load_gather(zp_v, [j16 >> 4])
            bi = (wi >> ((i16 & 15) * 2)) & 1
            bj = (wj >> ((j16 & 15) * 2 + 1)) & 1
            oki = bi & bj
            ok = oki == 1
            pos = off + plsc.cumsum(oki) - 1
            plsc.store_scatter(ik_v, [pos], i16, mask=ok)
            plsc.store_scatter(rk_v, [pos], r16, mask=ok)
            return off + jnp.sum(oki)

        ntot = lax.fori_loop(0, _VPC, _vec, 0)

        for t in range(_B // 16):
            ik_v[pl.ds(ntot + t * 16, 16)] = zeroi16
            rk_v[pl.ds(ntot + t * 16, 16)] = five16

        def _batch(b, c2):
            b0 = b * _B
            for v in range(_B // 16):
                rr = rk_v[pl.ds(b0 + v * 16, 16)]
                f = jnp.maximum(2.0 * (1.0 - rr * 0.2), 0.0)
                u = _log2(f)
                rowi = v * 16 + lanes
                for k in range(_FEATS):
                    y = jnp.exp(_CK[k] * u)
                    plsc.store_scatter(
                        fe_v, [rowi, jnp.full((16,), k, jnp.int32)], y
                    )
            pltpu.sync_copy(fe_v, acc_sh.at[ik_v.at[pl.ds(b0, _B)]], add=True)
            return c2

        nb = (ntot + _B - 1) >> 7
        lax.fori_loop(0, nb, _batch, 0)
        return carry

    lax.fori_loop(0, _CHUNKS, _chunk, 0)
    plsc.subcore_barrier()
    pltpu.sync_copy(
        acc_sh.at[pl.ds(a0, _ROWS_PER_TILE), :],
        out_hbm.at[cid, pl.ds(a0, _ROWS_PER_TILE), :],
    )


_sc_call = pl.kernel(
    _sc_body,
    out_type=jax.ShapeDtypeStruct((_NC, _ATOMS_PAD, _FEATS), jnp.float32),
    mesh=plsc.VectorSubcoreMesh(core_axis_name="c", subcore_axis_name="s"),
    compiler_params=pltpu.CompilerParams(
        needs_layout_passes=False, use_tc_tiling_on_sc=False
    ),
    scratch_types=[
        pltpu.VMEM((_ZP_WORDS,), jnp.int32),
        pltpu.VMEM((_E,), jnp.int32),
        pltpu.VMEM((_E,), jnp.int32),
        pltpu.VMEM((_E,), jnp.float32),
        pltpu.VMEM((_E + _B,), jnp.int32),
        pltpu.VMEM((_E + _B,), jnp.float32),
        pltpu.VMEM((_B, _FEATS), jnp.float32),
        pltpu.VMEM_SHARED((_ZP_WORDS,), jnp.int32),
        pltpu.VMEM_SHARED((_ATOMS_PAD, _FEATS), jnp.float32),
    ],
)


def _combine_body(p_ref, o_ref):
    o_ref[...] = p_ref[0] + p_ref[1]


def _combine(partial):
    p3 = partial.reshape(_NC, _ATOMS_PAD * _FEATS // 128, 128)
    out = pl.pallas_call(
        _combine_body,
        out_shape=jax.ShapeDtypeStruct((_ATOMS_PAD * _FEATS // 128, 128), jnp.float32),
    )(p3)
    return out.reshape(_ATOMS_PAD, _FEATS)[:_N_ATOMS]


def kernel(r, neighbour_list, Z):
    z_pad = jnp.concatenate([Z, jnp.zeros((_ATOMS_PAD - _N_ATOMS,), jnp.int32)])
    partial = _sc_call(r, neighbour_list.reshape(2 * _N_EDGES), z_pad)
    return _combine(partial)


# speedup vs baseline: 248.2711x; 1.1809x over previous
import math

import jax
import jax.numpy as jnp
from jax import lax
from jax.experimental import pallas as pl
from jax.experimental.pallas import tpu as pltpu
from jax.experimental.pallas import tpu_sc as plsc

_N_ATOMS = 100000
_N_EDGES = 6400000
_FEATS = 8
_NC = 2
_NS = 16
_NW = _NC * _NS
_PER_W = _N_EDGES // _NW
_E = 1600
_CHUNKS = _PER_W // _E
_VPC = _E // 16
_B = 128
_ATOMS_PAD = 100352
_ROWS_PER_TILE = _ATOMS_PAD // _NS
_ZP_WORDS = _ATOMS_PAD // 16
_ZPW_PER_TILE = _ZP_WORDS // _NS

_BETA = 4.0 ** (1.0 / 7.0)
_CK = [2.0 * _BETA**k * math.log(2.0) for k in range(_FEATS)]
_LOG2C = 2.0 / math.log(2.0)


def _vgather(v, idx):
    return lax.gather(
        v,
        idx[:, None],
        lax.GatherDimensionNumbers(
            offset_dims=(), collapsed_slice_dims=(0,), start_index_map=(0,)
        ),
        (1,),
        mode=lax.GatherScatterMode.PROMISE_IN_BOUNDS,
    )


def _log2(f):
    bits = plsc.bitcast(f, jnp.int32)
    e = ((bits >> 23) - 127).astype(jnp.float32)
    m = plsc.bitcast((bits & 0x7FFFFF) | 0x3F800000, jnp.float32)
    t = (m - 1.0) / (m + 1.0)
    t2 = t * t
    return e + _LOG2C * (
        t * (1.0 + t2 * (1.0 / 3.0 + t2 * (0.2 + t2 * (1.0 / 7.0))))
    )


def _sc_body(
    r_hbm, nl_hbm, z_hbm, out_hbm,
    zp_v, i_v, j_v, r_v, ik_v, rk_v, fe_v, zp_sh, acc_sh,
):
    cid = lax.axis_index("c")
    sid = lax.axis_index("s")
    wid = cid * _NS + sid
    lanes = lax.iota(jnp.int32, 16)
    zero16 = jnp.zeros((16,), jnp.float32)
    zeroi16 = lanes >> 4
    five16 = zero16 + 5.0
    lane0 = lanes == 0

    a0 = sid * _ROWS_PER_TILE

    for c in range(4):
        cs = _E if c < 3 else _ROWS_PER_TILE - 3 * _E
        pltpu.sync_copy(z_hbm.at[pl.ds(a0 + c * _E, cs)], i_v.at[pl.ds(0, cs)])

        def _pack(t, carry, _c=c):
            z16 = i_v[pl.ds(t * 16, 16)]
            code = jnp.where(z16 == 1, 1, 0) | jnp.where(z16 == 2, 2, 0)
            s = code << (lanes * 2)
            s = s | _vgather(s, lanes ^ 1)
            s = s | _vgather(s, lanes ^ 2)
            s = s | _vgather(s, lanes ^ 4)
            s = s | _vgather(s, lanes ^ 8)
            w = (a0 + _c * _E + t * 16) >> 4
            plsc.store_scatter(zp_v, [zeroi16 + w], s, mask=lane0)
            return carry

        lax.fori_loop(0, cs // 16, _pack, 0)

    pltpu.sync_copy(
        zp_v.at[pl.ds(sid * _ZPW_PER_TILE, _ZPW_PER_TILE)],
        zp_sh.at[pl.ds(sid * _ZPW_PER_TILE, _ZPW_PER_TILE)],
    )

    def _zero(t, carry):
        n = t * 16 + lanes
        plsc.store_scatter(fe_v, [n >> 3, n & 7], zero16)
        return carry

    lax.fori_loop(0, (_B * _FEATS) // 16, _zero, 0)
    for t in range(_ROWS_PER_TILE // _B):
        pltpu.sync_copy(fe_v, acc_sh.at[pl.ds(a0 + t * _B, _B), :])

    plsc.subcore_barrier()
    pltpu.sync_copy(zp_sh, zp_v)

    def _chunk(c, carry):
        base = wid * _PER_W + c * _E
        pltpu.sync_copy(nl_hbm.at[pl.ds(base, _E)], i_v)
        pltpu.sync_copy(nl_hbm.at[pl.ds(_N_EDGES + base, _E)], j_v)
        pltpu.sync_copy(r_hbm.at[pl.ds(base, _E)], r_v)

        splat15 = zeroi16 + 15

        def _vec(o, off_vec):
            datas = []
            for q in range(4):
                s = (o * 4 + q) * 16
                i16 = i_v[pl.ds(s, 16)]
                j16 = j_v[pl.ds(s, 16)]
                r16 = r_v[pl.ds(s, 16)]
                wi = plsc.load_gather(zp_v, [i16 >> 4])
                wj = plsc.load_gather(zp_v, [j16 >> 4])
                bi = (wi >> ((i16 & 15) * 2)) & 1
                bj = (wj >> ((j16 & 15) * 2 + 1)) & 1
                oki = bi & bj
                cs = plsc.cumsum(oki)
                datas.append((i16, r16, oki == 1, cs))
            for i16, r16, ok, cs in datas:
                pos = off_vec + cs - 1
                plsc.store_scatter(ik_v, [pos], i16, mask=ok)
                plsc.store_scatter(rk_v, [pos], r16, mask=ok)
                off_vec = off_vec + _vgather(cs, splat15)
            return off_vec

        off_vec = lax.fori_loop(0, _VPC // 4, _vec, zeroi16)
        ntot = jnp.sum(off_vec) >> 4

        for t in range(_B // 16):
            ik_v[pl.ds(ntot + t * 16, 16)] = zeroi16
            rk_v[pl.ds(ntot + t * 16, 16)] = five16

        def _batch(b, c2):
            b0 = b * _B
            for v in range(_B // 16):
                rr = rk_v[pl.ds(b0 + v * 16, 16)]
                f = jnp.maximum(2.0 * (1.0 - rr * 0.2), 0.0)
                u = _log2(f)
                rowi = v * 16 + lanes
                for k in range(_FEATS):
                    y = jnp.exp(_CK[k] * u)
                    plsc.store_scatter(
                        fe_v, [rowi, jnp.full((16,), k, jnp.int32)], y
                    )
            pltpu.sync_copy(fe_v, acc_sh.at[ik_v.at[pl.ds(b0, _B)]], add=True)
            return c2

        nb = (ntot + _B - 1) >> 7
        lax.fori_loop(0, nb, _batch, 0)
        return carry

    lax.fori_loop(0, _CHUNKS, _chunk, 0)
    plsc.subcore_barrier()
    pltpu.sync_copy(
        acc_sh.at[pl.ds(a0, _ROWS_PER_TILE), :],
        out_hbm.at[cid, pl.ds(a0, _ROWS_PER_TILE), :],
    )


_sc_call = pl.kernel(
    _sc_body,
    out_type=jax.ShapeDtypeStruct((_NC, _ATOMS_PAD, _FEATS), jnp.float32),
    mesh=plsc.VectorSubcoreMesh(core_axis_name="c", subcore_axis_name="s"),
    compiler_params=pltpu.CompilerParams(
        needs_layout_passes=False, use_tc_tiling_on_sc=False
    ),
    scratch_types=[
        pltpu.VMEM((_ZP_WORDS,), jnp.int32),
        pltpu.VMEM((_E,), jnp.int32),
        pltpu.VMEM((_E,), jnp.int32),
        pltpu.VMEM((_E,), jnp.float32),
        pltpu.VMEM((_E + _B,), jnp.int32),
        pltpu.VMEM((_E + _B,), jnp.float32),
        pltpu.VMEM((_B, _FEATS), jnp.float32),
        pltpu.VMEM_SHARED((_ZP_WORDS,), jnp.int32),
        pltpu.VMEM_SHARED((_ATOMS_PAD, _FEATS), jnp.float32),
    ],
)


def _combine_body(p_ref, o_ref):
    o_ref[...] = p_ref[0] + p_ref[1]


def _combine(partial):
    p3 = partial.reshape(_NC, _ATOMS_PAD * _FEATS // 128, 128)
    out = pl.pallas_call(
        _combine_body,
        out_shape=jax.ShapeDtypeStruct((_ATOMS_PAD * _FEATS // 128, 128), jnp.float32),
    )(p3)
    return out.reshape(_ATOMS_PAD, _FEATS)[:_N_ATOMS]


def kernel(r, neighbour_list, Z):
    z_pad = jnp.concatenate([Z, jnp.zeros((_ATOMS_PAD - _N_ATOMS,), jnp.int32)])
    partial = _sc_call(r, neighbour_list.reshape(2 * _N_EDGES), z_pad)
    return _combine(partial)


# speedup vs baseline: 437.4191x; 1.7619x over previous
import math

import jax
import jax.numpy as jnp
from jax import lax
from jax.experimental import pallas as pl
from jax.experimental.pallas import tpu as pltpu
from jax.experimental.pallas import tpu_sc as plsc

_N_ATOMS = 100000
_N_EDGES = 6400000
_FEATS = 8
_NC = 2
_NS = 16
_NW = _NC * _NS
_PER_W = _N_EDGES // _NW
_E = 1600
_CHUNKS = _PER_W // _E
_VPC = _E // 16
_B = 128
_ATOMS_PAD = 100352
_ROWS_PER_TILE = _ATOMS_PAD // _NS
_ZP_WORDS = _ATOMS_PAD // 16
_ZPW_PER_TILE = _ZP_WORDS // _NS

_BETA = 4.0 ** (1.0 / 7.0)
_CK = [2.0 * _BETA**k * math.log(2.0) for k in range(_FEATS)]
_LOG2C = 2.0 / math.log(2.0)


def _vgather(v, idx):
    return lax.gather(
        v,
        idx[:, None],
        lax.GatherDimensionNumbers(
            offset_dims=(), collapsed_slice_dims=(0,), start_index_map=(0,)
        ),
        (1,),
        mode=lax.GatherScatterMode.PROMISE_IN_BOUNDS,
    )


def _log2(f):
    bits = plsc.bitcast(f, jnp.int32)
    e = ((bits >> 23) - 127).astype(jnp.float32)
    m = plsc.bitcast((bits & 0x7FFFFF) | 0x3F800000, jnp.float32)
    t = (m - 1.0) / (m + 1.0)
    t2 = t * t
    return e + _LOG2C * (
        t * (1.0 + t2 * (1.0 / 3.0 + t2 * (0.2 + t2 * (1.0 / 7.0))))
    )


def _sc_body(
    r_hbm, nl_hbm, z_hbm, out_hbm,
    zp_v, ij_a, ij_b, r_a, r_b, ik_v, rk_v, fe_v, zp_sh, acc_sh,
    sem_a, sem_b,
):
    cid = lax.axis_index("c")
    sid = lax.axis_index("s")
    wid = cid * _NS + sid
    lanes = lax.iota(jnp.int32, 16)
    zero16 = jnp.zeros((16,), jnp.float32)
    zeroi16 = lanes >> 4
    five16 = zero16 + 5.0
    lane0 = lanes == 0
    splat15 = zeroi16 + 15

    a0 = sid * _ROWS_PER_TILE

    for c in range(4):
        cs = _E if c < 3 else _ROWS_PER_TILE - 3 * _E
        pltpu.sync_copy(z_hbm.at[pl.ds(a0 + c * _E, cs)], ik_v.at[pl.ds(0, cs)])

        def _pack(t, carry, _c=c):
            z16 = ik_v[pl.ds(t * 16, 16)]
            code = jnp.where(z16 == 1, 1, 0) | jnp.where(z16 == 2, 2, 0)
            s = code << (lanes * 2)
            s = s | _vgather(s, lanes ^ 1)
            s = s | _vgather(s, lanes ^ 2)
            s = s | _vgather(s, lanes ^ 4)
            s = s | _vgather(s, lanes ^ 8)
            w = (a0 + _c * _E + t * 16) >> 4
            plsc.store_scatter(zp_v, [zeroi16 + w], s, mask=lane0)
            return carry

        lax.fori_loop(0, cs // 16, _pack, 0)

    pltpu.sync_copy(
        zp_v.at[pl.ds(sid * _ZPW_PER_TILE, _ZPW_PER_TILE)],
        zp_sh.at[pl.ds(sid * _ZPW_PER_TILE, _ZPW_PER_TILE)],
    )

    def _zero(t, carry):
        n = t * 16 + lanes
        plsc.store_scatter(fe_v, [n >> 3, n & 7], zero16)
        return carry

    lax.fori_loop(0, (_B * _FEATS) // 16, _zero, 0)
    for t in range(_ROWS_PER_TILE // _B):
        pltpu.sync_copy(fe_v, acc_sh.at[pl.ds(a0 + t * _B, _B), :])

    plsc.subcore_barrier()
    pltpu.sync_copy(zp_sh, zp_v)

    ebase = wid * _PER_W

    def _start(c, ij_ref, r_ref, sem):
        hij = pltpu.async_copy(
            nl_hbm.at[:, pl.ds(ebase + c * _E, _E)], ij_ref, sem
        )
        hr = pltpu.async_copy(r_hbm.at[pl.ds(ebase + c * _E, _E)], r_ref, sem)
        return hij, hr

    def _scatter_batch(b0):
        for v in range(_B // 16):
            rr = rk_v[pl.ds(b0 + v * 16, 16)]
            f = jnp.maximum(2.0 * (1.0 - rr * 0.2), 0.0)
            u = _log2(f)
            rowi = v * 16 + lanes
            for k in range(_FEATS):
                y = jnp.exp(_CK[k] * u)
                plsc.store_scatter(
                    fe_v, [rowi, jnp.full((16,), k, jnp.int32)], y
                )
        pltpu.sync_copy(fe_v, acc_sh.at[ik_v.at[pl.ds(b0, _B)]], add=True)

    def _process(c, ij_ref, r_ref, off_vec):
        del c
        def _vec(o, ov):
            datas = []
            for q in range(4):
                s = (o * 4 + q) * 16
                i16 = ij_ref[0, pl.ds(s, 16)]
                j16 = ij_ref[1, pl.ds(s, 16)]
                r16 = r_ref[pl.ds(s, 16)]
                wi = plsc.load_gather(zp_v, [i16 >> 4])
                wj = plsc.load_gather(zp_v, [j16 >> 4])
                bi = (wi >> ((i16 & 15) * 2)) & 1
                bj = (wj >> ((j16 & 15) * 2 + 1)) & 1
                oki = bi & bj
                cs = plsc.cumsum(oki)
                datas.append((i16, r16, oki == 1, cs))
            for i16, r16, ok, cs in datas:
                pos = ov + cs - 1
                plsc.store_scatter(ik_v, [pos], i16, mask=ok)
                plsc.store_scatter(rk_v, [pos], r16, mask=ok)
                ov = ov + _vgather(cs, splat15)
            return ov

        off_vec = lax.fori_loop(0, _VPC // 4, _vec, off_vec)
        ntot = jnp.sum(off_vec) >> 4
        nb = ntot >> 7

        def _batch(b, c2):
            _scatter_batch(b * _B)
            return c2

        lax.fori_loop(0, nb, _batch, 0)
        for t in range(_B // 16):
            ik_v[pl.ds(t * 16, 16)] = ik_v[pl.ds(nb * _B + t * 16, 16)]
            rk_v[pl.ds(t * 16, 16)] = rk_v[pl.ds(nb * _B + t * 16, 16)]
        return off_vec - (nb << 7)

    ha = _start(0, ij_a, r_a, sem_a)
    for h in ha:
        h.wait()

    def _pair(h, off_vec):
        c0 = h * 2
        hb = _start(c0 + 1, ij_b, r_b, sem_b)
        off_vec = _process(c0, ij_a, r_a, off_vec)
        for hh in hb:
            hh.wait()
        ha2 = _start(c0 + 2, ij_a, r_a, sem_a)
        off_vec = _process(c0 + 1, ij_b, r_b, off_vec)
        for hh in ha2:
            hh.wait()
        return off_vec

    off_vec = lax.fori_loop(0, (_CHUNKS - 1) // 2, _pair, zeroi16)
    off_vec = _process(_CHUNKS - 1, ij_a, r_a, off_vec)

    rem = jnp.sum(off_vec) >> 4
    for t in range(_B // 16):
        ik_v[pl.ds(rem + t * 16, 16)] = zeroi16
        rk_v[pl.ds(rem + t * 16, 16)] = five16
    _scatter_batch(0)

    plsc.subcore_barrier()
    pltpu.sync_copy(
        acc_sh.at[pl.ds(a0, _ROWS_PER_TILE), :],
        out_hbm.at[cid, pl.ds(a0, _ROWS_PER_TILE), :],
    )


_sc_call = pl.kernel(
    _sc_body,
    out_type=jax.ShapeDtypeStruct((_NC, _ATOMS_PAD, _FEATS), jnp.float32),
    mesh=plsc.VectorSubcoreMesh(core_axis_name="c", subcore_axis_name="s"),
    compiler_params=pltpu.CompilerParams(
        needs_layout_passes=False, use_tc_tiling_on_sc=False
    ),
    scratch_types=[
        pltpu.VMEM((_ZP_WORDS,), jnp.int32),
        pltpu.VMEM((2, _E), jnp.int32),
        pltpu.VMEM((2, _E), jnp.int32),
        pltpu.VMEM((_E,), jnp.float32),
        pltpu.VMEM((_E,), jnp.float32),
        pltpu.VMEM((_E + 2 * _B,), jnp.int32),
        pltpu.VMEM((_E + 2 * _B,), jnp.float32),
        pltpu.VMEM((_B, _FEATS), jnp.float32),
        pltpu.VMEM_SHARED((_ZP_WORDS,), jnp.int32),
        pltpu.VMEM_SHARED((_ATOMS_PAD, _FEATS), jnp.float32),
        pltpu.SemaphoreType.DMA,
        pltpu.SemaphoreType.DMA,
    ],
)


def _combine_body(p_ref, o_ref):
    o_ref[...] = p_ref[0] + p_ref[1]


def _combine(partial):
    p3 = partial.reshape(_NC, _ATOMS_PAD * _FEATS // 128, 128)
    out = pl.pallas_call(
        _combine_body,
        out_shape=jax.ShapeDtypeStruct((_ATOMS_PAD * _FEATS // 128, 128), jnp.float32),
    )(p3)
    return out.reshape(_ATOMS_PAD, _FEATS)[:_N_ATOMS]


def kernel(r, neighbour_list, Z):
    z_pad = jnp.concatenate([Z, jnp.zeros((_ATOMS_PAD - _N_ATOMS,), jnp.int32)])
    partial = _sc_call(r, neighbour_list, z_pad)
    return _combine(partial)
